# baseline (speedup 1.0000x reference)
import functools

import jax
import jax.numpy as jnp
from jax import lax
from jax.experimental import pallas as pl
from jax.experimental.pallas import tpu as pltpu
from jax.experimental.pallas import tpu_sc as plsc

_SIGMA = 0.1
_N = 4096
_L = 16
_NC = 2
_NS = 16
_NW = _NC * _NS
_ROWS = _N // _NW
_AG = 8
_NB = _N // _L


def _bcast_lane(vec, idxv):
    return lax.gather(
        vec,
        idxv[:, None],
        lax.GatherDimensionNumbers(
            offset_dims=(), collapsed_slice_dims=(0,), start_index_map=(0,)
        ),
        (1,),
        indices_are_sorted=False,
        unique_indices=False,
        mode=lax.GatherScatterMode.PROMISE_IN_BOUNDS,
    )


def _bcast_lane_idx(vec, idxv):
    return lax.gather(
        vec,
        idxv[:, None],
        lax.GatherDimensionNumbers(
            offset_dims=(), collapsed_slice_dims=(0,), start_index_map=(0,)
        ),
        (1,),
        indices_are_sorted=False,
        unique_indices=False,
        mode=lax.GatherScatterMode.PROMISE_IN_BOUNDS,
    )


def _cindex_sc_kernel(t_hbm, e_hbm, f_hbm, out_hbm, t_v, e_v, f_v, o_v):
    wid = lax.axis_index("s") * _NC + lax.axis_index("c")
    pltpu.sync_copy(t_hbm, t_v)
    pltpu.sync_copy(e_hbm, e_v)
    pltpu.sync_copy(f_hbm, f_v)

    def exp_body(i, c):
        sl = pl.ds(i * _L, _L)
        e_v[sl] = jnp.exp(e_v[sl] * (1.0 / _SIGMA))
        return c

    lax.fori_loop(0, _NB, exp_body, 0)

    base = wid * _ROWS
    zero = jnp.zeros((_L,), jnp.float32)

    def a_body(bi, carry):
        sl_a = pl.ds(base + bi * _L, _L)
        ta_blk = t_v[sl_a]
        ea_blk = e_v[sl_a]

        def k_body(kg, carry2):
            splats = []
            for j in range(_AG):
                idxv = jnp.full((_L,), kg * _AG + j, jnp.int32)
                splats.append((_bcast_lane(ta_blk, idxv), _bcast_lane(ea_blk, idxv)))

            def b_body(c, carry3):
                accn3, accd3 = carry3
                sl = pl.ds(c * _L, _L)
                tb = t_v[sl]
                eb = e_v[sl]
                fb = f_v[sl]
                for ta, ea in splats:
                    mf = jnp.where(ta > tb, fb, 0.0)
                    q = ea / (ea + eb)
                    accn3 = accn3 + q * mf
                    accd3 = accd3 + mf
                return accn3, accd3

            return lax.fori_loop(0, _NB, b_body, carry2)

        return lax.fori_loop(0, _L // _AG, k_body, carry)

    accn, accd = lax.fori_loop(0, _ROWS // _L, a_body, (zero, zero))
    o_v[pl.ds(0, _L)] = accn
    o_v[pl.ds(_L, _L)] = accd
    pltpu.sync_copy(o_v, out_hbm.at[wid])


@jax.jit
def kernel(times, scores, events):
    mesh = plsc.VectorSubcoreMesh(core_axis_name="c", subcore_axis_name="s")
    partials = pl.kernel(
        _cindex_sc_kernel,
        mesh=mesh,
        out_type=jax.ShapeDtypeStruct((_NW, 2 * _L), jnp.float32),
        scratch_types=[
            pltpu.VMEM((_N,), jnp.float32),
            pltpu.VMEM((_N,), jnp.float32),
            pltpu.VMEM((_N,), jnp.float32),
            pltpu.VMEM((2 * _L,), jnp.float32),
        ],
    )(times, scores, events.astype(jnp.float32))
    num = partials[:, :_L].sum()
    den = partials[:, _L:].sum()
    return num / (den + 1.0)


# speedup vs baseline: 1.0120x; 1.0120x over previous
import functools

import jax
import jax.numpy as jnp
from jax import lax
from jax.experimental import pallas as pl
from jax.experimental.pallas import tpu as pltpu
from jax.experimental.pallas import tpu_sc as plsc

_SIGMA = 0.1
_N = 4096
_L = 16
_NC = 2
_NS = 16
_NW = _NC * _NS
_ROWS = _N // _NW
_AG = 4
_NB = _N // _L


def _bcast_lane(vec, idxv):
    return lax.gather(
        vec,
        idxv[:, None],
        lax.GatherDimensionNumbers(
            offset_dims=(), collapsed_slice_dims=(0,), start_index_map=(0,)
        ),
        (1,),
        indices_are_sorted=False,
        unique_indices=False,
        mode=lax.GatherScatterMode.PROMISE_IN_BOUNDS,
    )


def _bcast_lane_idx(vec, idxv):
    return lax.gather(
        vec,
        idxv[:, None],
        lax.GatherDimensionNumbers(
            offset_dims=(), collapsed_slice_dims=(0,), start_index_map=(0,)
        ),
        (1,),
        indices_are_sorted=False,
        unique_indices=False,
        mode=lax.GatherScatterMode.PROMISE_IN_BOUNDS,
    )


def _cindex_sc_kernel(t_hbm, e_hbm, f_hbm, out_hbm, t_v, e_v, f_v, o_v):
    wid = lax.axis_index("s") * _NC + lax.axis_index("c")
    pltpu.sync_copy(t_hbm, t_v)
    pltpu.sync_copy(e_hbm, e_v)
    pltpu.sync_copy(f_hbm, f_v)

    def exp_body(i, c):
        sl = pl.ds(i * _L, _L)
        e_v[sl] = jnp.exp(e_v[sl] * (1.0 / _SIGMA))
        return c

    lax.fori_loop(0, _NB, exp_body, 0)

    base = wid * _ROWS
    zero = jnp.zeros((_L,), jnp.float32)

    def a_body(bi, carry):
        sl_a = pl.ds(base + bi * _L, _L)
        ta_blk = t_v[sl_a]
        ea_blk = e_v[sl_a]

        def k_body(kg, carry2):
            splats = []
            for j in range(_AG):
                idxv = jnp.full((_L,), kg * _AG + j, jnp.int32)
                splats.append((_bcast_lane(ta_blk, idxv), _bcast_lane(ea_blk, idxv)))

            def b_body(c, carry3):
                accn3, accd3 = carry3
                sl = pl.ds(c * _L, _L)
                tb = t_v[sl]
                eb = e_v[sl]
                fb = f_v[sl]
                for ta, ea in splats:
                    mf = jnp.where(ta > tb, fb, 0.0)
                    q = ea / (ea + eb)
                    accn3 = accn3 + q * mf
                    accd3 = accd3 + mf
                return accn3, accd3

            return lax.fori_loop(0, _NB, b_body, carry2, unroll=4)

        return lax.fori_loop(0, _L // _AG, k_body, carry)

    accn, accd = lax.fori_loop(0, _ROWS // _L, a_body, (zero, zero))
    o_v[pl.ds(0, _L)] = accn
    o_v[pl.ds(_L, _L)] = accd
    pltpu.sync_copy(o_v, out_hbm.at[wid])


@jax.jit
def kernel(times, scores, events):
    mesh = plsc.VectorSubcoreMesh(core_axis_name="c", subcore_axis_name="s")
    partials = pl.kernel(
        _cindex_sc_kernel,
        mesh=mesh,
        out_type=jax.ShapeDtypeStruct((_NW, 2 * _L), jnp.float32),
        scratch_types=[
            pltpu.VMEM((_N,), jnp.float32),
            pltpu.VMEM((_N,), jnp.float32),
            pltpu.VMEM((_N,), jnp.float32),
            pltpu.VMEM((2 * _L,), jnp.float32),
        ],
    )(times, scores, events.astype(jnp.float32))
    num = partials[:, :_L].sum()
    den = partials[:, _L:].sum()
    return num / (den + 1.0)


# speedup vs baseline: 1.6231x; 1.6038x over previous
import functools

import jax
import jax.numpy as jnp
from jax import lax
from jax.experimental import pallas as pl
from jax.experimental.pallas import tpu as pltpu
from jax.experimental.pallas import tpu_sc as plsc

_SIGMA = 0.1
_N = 4096
_L = 16
_NC = 2
_NS = 16
_NW = _NC * _NS
_AG = 4
_NB = _N // _L

_SPLIT = 1024
_SC_ROWS = _SPLIT // _NW
_BR = 256
_TC_PROGS = (_N - _SPLIT) // _BR


def _bcast_lane(vec, idxv):
    return lax.gather(
        vec,
        idxv[:, None],
        lax.GatherDimensionNumbers(
            offset_dims=(), collapsed_slice_dims=(0,), start_index_map=(0,)
        ),
        (1,),
        indices_are_sorted=False,
        unique_indices=False,
        mode=lax.GatherScatterMode.PROMISE_IN_BOUNDS,
    )


def _cindex_sc_kernel(t_hbm, e_hbm, f_hbm, out_hbm, t_v, e_v, f_v, o_v):
    wid = lax.axis_index("s") * _NC + lax.axis_index("c")
    pltpu.sync_copy(t_hbm, t_v)
    pltpu.sync_copy(e_hbm, e_v)
    pltpu.sync_copy(f_hbm, f_v)

    def exp_body(i, c):
        sl = pl.ds(i * _L, _L)
        e_v[sl] = jnp.exp(e_v[sl] * (1.0 / _SIGMA))
        return c

    lax.fori_loop(0, _NB, exp_body, 0)

    base = wid * _SC_ROWS
    zero = jnp.zeros((_L,), jnp.float32)

    def a_body(bi, carry):
        sl_a = pl.ds(base + bi * _L, _L)
        ta_blk = t_v[sl_a]
        ea_blk = e_v[sl_a]

        def k_body(kg, carry2):
            splats = []
            for j in range(_AG):
                idxv = jnp.full((_L,), kg * _AG + j, jnp.int32)
                splats.append((_bcast_lane(ta_blk, idxv), _bcast_lane(ea_blk, idxv)))

            def b_body(c, carry3):
                accn3, accd3 = carry3
                sl = pl.ds(c * _L, _L)
                tb = t_v[sl]
                eb = e_v[sl]
                fb = f_v[sl]
                for ta, ea in splats:
                    mf = jnp.where(ta > tb, fb, 0.0)
                    q = ea / (ea + eb)
                    accn3 = accn3 + q * mf
                    accd3 = accd3 + mf
                return accn3, accd3

            return lax.fori_loop(0, _NB, b_body, carry2, unroll=4)

        return lax.fori_loop(0, _L // _AG, k_body, carry)

    accn, accd = lax.fori_loop(0, _SC_ROWS // _L, a_body, (zero, zero))
    o_v[pl.ds(0, _L)] = accn
    o_v[pl.ds(_L, _L)] = accd
    pltpu.sync_copy(o_v, out_hbm.at[wid])


def _cindex_tc_kernel(tcol_ref, scol_ref, trow_ref, srow_ref, frow_ref, out_ref):
    ta = tcol_ref[...]
    ea = jnp.exp(scol_ref[...] * (1.0 / _SIGMA))
    tb = trow_ref[...]
    eb = jnp.exp(srow_ref[...] * (1.0 / _SIGMA))
    fb = frow_ref[...]
    mf = jnp.where(ta > tb, fb, 0.0)
    q = ea / (ea + eb)
    num = jnp.sum(q * mf).reshape(1, 1, 1)
    den = jnp.sum(mf).reshape(1, 1, 1)
    out_ref[...] = jnp.concatenate([num, den], axis=2)


@jax.jit
def kernel(times, scores, events):
    eventsf = events.astype(jnp.float32)

    mesh = plsc.VectorSubcoreMesh(core_axis_name="c", subcore_axis_name="s")
    sc_partials = pl.kernel(
        _cindex_sc_kernel,
        mesh=mesh,
        out_type=jax.ShapeDtypeStruct((_NW, 2 * _L), jnp.float32),
        scratch_types=[
            pltpu.VMEM((_N,), jnp.float32),
            pltpu.VMEM((_N,), jnp.float32),
            pltpu.VMEM((_N,), jnp.float32),
            pltpu.VMEM((2 * _L,), jnp.float32),
        ],
    )(times, scores, eventsf)

    tcol = times[_SPLIT:].reshape(-1, 1)
    scol = scores[_SPLIT:].reshape(-1, 1)
    trow = times.reshape(1, _N)
    srow = scores.reshape(1, _N)
    frow = eventsf.reshape(1, _N)

    tc_partials = pl.pallas_call(
        _cindex_tc_kernel,
        grid=(_TC_PROGS,),
        in_specs=[
            pl.BlockSpec((_BR, 1), lambda i: (i, 0)),
            pl.BlockSpec((_BR, 1), lambda i: (i, 0)),
            pl.BlockSpec((1, _N), lambda i: (0, 0)),
            pl.BlockSpec((1, _N), lambda i: (0, 0)),
            pl.BlockSpec((1, _N), lambda i: (0, 0)),
        ],
        out_specs=pl.BlockSpec((1, 1, 2), lambda i: (i, 0, 0)),
        out_shape=jax.ShapeDtypeStruct((_TC_PROGS, 1, 2), jnp.float32),
        compiler_params=pltpu.CompilerParams(
            dimension_semantics=("parallel",),
        ),
    )(tcol, scol, trow, srow, frow)

    num = sc_partials[:, :_L].sum() + tc_partials[:, 0, 0].sum()
    den = sc_partials[:, _L:].sum() + tc_partials[:, 0, 1].sum()
    return num / (den + 1.0)
